# baseline (speedup 1.0000x reference)
import functools

import jax
import jax.numpy as jnp
from jax import lax
from jax.experimental import pallas as pl
from jax.experimental.pallas import tpu as pltpu
from jax.experimental.pallas import tpu_sc as plsc

_B = 16
_N = 16384
_H = _N // 2
_NOUT = 1360
_L = 16
_ROW = 2 * _L


def _fps_body(xs, ys, zs, out, shared, xv, yv, zv, dist,
              exw, exr, s2x, s2y, s2z, s3x, s3y, s3z, s4x, s4y, s4z, outv):
    c = lax.axis_index("c")
    s = lax.axis_index("s")
    cloud = c * 8 + s // 2
    half = s % 2

    pltpu.sync_copy(xs.at[cloud], xv)
    pltpu.sync_copy(ys.at[cloud], yv)
    pltpu.sync_copy(zs.at[cloud], zv)

    lane = lax.iota(jnp.int32, _L)
    lane0 = lane == 0
    big = jnp.full((_L,), 1e10, jnp.float32)

    def scan_half(px, py, pz, pt_base, n_local, cx, cy, cz):
        nacc = 8 if n_local >= 2048 else (4 if n_local >= 64 else 1)
        unroll = 1 if nacc == 8 else 2
        carry0 = (
            tuple(jnp.full((_L,), -1.0, jnp.float32) for _ in range(nacc)),
            tuple(jnp.zeros((_L,), jnp.int32) for _ in range(nacc)),
        )

        def scan(j, carry):
            avs, ais = carry
            navs, nais = [], []
            for k in range(nacc):
                b0 = j + k * _L
                dx = px[pl.ds(pt_base + b0, _L)] - cx
                dy = py[pl.ds(pt_base + b0, _L)] - cy
                dz = pz[pl.ds(pt_base + b0, _L)] - cz
                d = dx * dx + dy * dy + dz * dz
                dn = jnp.minimum(dist[pl.ds(b0, _L)], d)
                dist[pl.ds(b0, _L)] = dn
                m = dn > avs[k]
                navs.append(jnp.where(m, dn, avs[k]))
                nais.append(jnp.where(m, lane + (pt_base + b0), ais[k]))
            return tuple(navs), tuple(nais)

        avs, ais = plsc.parallel_loop(
            0, n_local, step=nacc * _L, unroll=unroll, carry=carry0)(scan)
        em = avs[0]
        for k in range(1, nacc):
            em = jnp.maximum(em, avs[k])
        mgv = jnp.full((_L,), jnp.max(em))
        cm = jnp.full((_L,), _N, jnp.int32)
        for k in range(nacc):
            cm = jnp.minimum(cm, jnp.where(avs[k] == mgv, ais[k], _N))
        nxtv = jnp.full((_L,), jnp.min(cm))
        return mgv, nxtv

    def init_dist(n_local):
        def init(j, carry):
            dist[pl.ds(j * _L, _L)] = big
            return carry
        lax.fori_loop(0, n_local // _L, init, 0, unroll=4)

    init_dist(_H)
    pt_base = half * _H
    myslot = s * 2 * _ROW
    pslot = (s ^ 1) * 2 * _ROW

    def stage1_iter(i, farv):
        cx = plsc.load_gather(xv, [farv])
        cy = plsc.load_gather(yv, [farv])
        cz = plsc.load_gather(zv, [farv])

        @pl.when(half == 0)
        def _():
            ivs = jnp.full((_L,), i, jnp.int32)
            plsc.store_scatter(outv, [ivs], farv, mask=lane0)
            plsc.store_scatter(s2x, [ivs], cx, mask=lane0)
            plsc.store_scatter(s2y, [ivs], cy, mask=lane0)
            plsc.store_scatter(s2z, [ivs], cz, mask=lane0)

        mgv, nxtv = scan_half(xv, yv, zv, pt_base, _H, cx, cy, cz)

        exw[pl.ds(0, _L)] = plsc.bitcast(mgv, jnp.int32)
        exw[pl.ds(_L, _L)] = nxtv
        parity = (i % 2) * _ROW
        pltpu.sync_copy(exw, shared.at[pl.ds(myslot + parity, _ROW)])
        plsc.subcore_barrier()
        pltpu.sync_copy(shared.at[pl.ds(pslot + parity, _ROW)], exr)
        pmv = plsc.bitcast(exr[pl.ds(0, _L)], jnp.float32)
        piv = exr[pl.ds(_L, _L)]
        take = (pmv > mgv) | ((pmv == mgv) & (piv < nxtv))
        return jnp.where(take, piv, nxtv)

    lax.fori_loop(0, 1024, stage1_iter, jnp.zeros((_L,), jnp.int32))

    @pl.when(half == 0)
    def _():
        def run_stage(px, py, pz, n, npoint, off, prev_off, sel):
            init_dist(n)

            def one_iter(i, farv):
                cx = plsc.load_gather(px, [farv])
                cy = plsc.load_gather(py, [farv])
                cz = plsc.load_gather(pz, [farv])
                iv = jnp.full((_L,), off + i, jnp.int32)
                val = plsc.load_gather(outv, [prev_off + farv])
                plsc.store_scatter(outv, [iv], val, mask=lane0)
                if sel is not None:
                    sx, sy, sz = sel
                    ivs = jnp.full((_L,), i, jnp.int32)
                    plsc.store_scatter(sx, [ivs], cx, mask=lane0)
                    plsc.store_scatter(sy, [ivs], cy, mask=lane0)
                    plsc.store_scatter(sz, [ivs], cz, mask=lane0)
                _, nxtv = scan_half(px, py, pz, 0, n, cx, cy, cz)
                return nxtv

            lax.fori_loop(0, npoint, one_iter, jnp.zeros((_L,), jnp.int32))

        run_stage(s2x, s2y, s2z, 1024, 256, 1024, 0, (s3x, s3y, s3z))
        run_stage(s3x, s3y, s3z, 256, 64, 1280, 1024, (s4x, s4y, s4z))
        run_stage(s4x, s4y, s4z, 64, 16, 1344, 1280, None)

        pltpu.sync_copy(outv, out.at[cloud])


@jax.jit
def kernel(x):
    xyz = x[..., :3]
    xs = xyz[..., 0]
    ys = xyz[..., 1]
    zs = xyz[..., 2]
    f = pl.kernel(
        _fps_body,
        out_type=jax.ShapeDtypeStruct((_B, _NOUT), jnp.int32),
        mesh=plsc.VectorSubcoreMesh(core_axis_name="c", subcore_axis_name="s"),
        compiler_params=pltpu.CompilerParams(needs_layout_passes=False),
        scratch_types=[
            pltpu.VMEM_SHARED((16 * 2 * _ROW,), jnp.int32),
            pltpu.VMEM((_N,), jnp.float32),
            pltpu.VMEM((_N,), jnp.float32),
            pltpu.VMEM((_N,), jnp.float32),
            pltpu.VMEM((_H,), jnp.float32),
            pltpu.VMEM((_ROW,), jnp.int32),
            pltpu.VMEM((_ROW,), jnp.int32),
            pltpu.VMEM((1024,), jnp.float32),
            pltpu.VMEM((1024,), jnp.float32),
            pltpu.VMEM((1024,), jnp.float32),
            pltpu.VMEM((256,), jnp.float32),
            pltpu.VMEM((256,), jnp.float32),
            pltpu.VMEM((256,), jnp.float32),
            pltpu.VMEM((64,), jnp.float32),
            pltpu.VMEM((64,), jnp.float32),
            pltpu.VMEM((64,), jnp.float32),
            pltpu.VMEM((_NOUT,), jnp.int32),
        ],
    )
    return f(xs, ys, zs)


# speedup vs baseline: 1.0688x; 1.0688x over previous
import jax
import jax.numpy as jnp
from jax import lax
from jax.experimental import pallas as pl
from jax.experimental.pallas import tpu as pltpu
from jax.experimental.pallas import tpu_sc as plsc

_B = 16
_N = 16384
_H = _N // 2
_NOUT = 1360
_L = 16
_ROW = 2 * _L


def _fps_body(xs, ys, zs, out, shared, xv, yv, zv, dist,
              exw, exr, s2x, s2y, s2z, s3x, s3y, s3z, s4x, s4y, s4z, outv):
    c = lax.axis_index("c")
    s = lax.axis_index("s")
    cloud = c * 8 + s // 2
    half = s % 2

    pltpu.sync_copy(xs.at[cloud], xv)
    pltpu.sync_copy(ys.at[cloud], yv)
    pltpu.sync_copy(zs.at[cloud], zv)

    lane = lax.iota(jnp.int32, _L)
    lane0 = lane == 0
    big = jnp.full((_L,), 1e10, jnp.float32)

    def scan_half(px, py, pz, pt_base, n_local, cx, cy, cz):
        nacc = 4 if n_local >= 64 else 1
        carry0 = (
            tuple(jnp.full((_L,), -1.0, jnp.float32) for _ in range(nacc)),
            tuple(jnp.zeros((_L,), jnp.int32) for _ in range(nacc)),
        )

        def scan(j, carry):
            avs, ais = carry
            navs, nais = [], []
            for k in range(nacc):
                b0 = j + k * _L
                dx = px[pl.ds(pt_base + b0, _L)] - cx
                dy = py[pl.ds(pt_base + b0, _L)] - cy
                dz = pz[pl.ds(pt_base + b0, _L)] - cz
                d = dx * dx + dy * dy + dz * dz
                dn = jnp.minimum(dist[pl.ds(b0, _L)], d)
                dist[pl.ds(b0, _L)] = dn
                m = dn > avs[k]
                navs.append(jnp.where(m, dn, avs[k]))
                nais.append(jnp.where(m, lane + (pt_base + b0), ais[k]))
            return tuple(navs), tuple(nais)

        avs, ais = plsc.parallel_loop(
            0, n_local, step=nacc * _L, unroll=2, carry=carry0)(scan)
        em = avs[0]
        for k in range(1, nacc):
            em = jnp.maximum(em, avs[k])
        mgv = jnp.full((_L,), jnp.max(em))
        cm = jnp.full((_L,), _N, jnp.int32)
        for k in range(nacc):
            cm = jnp.minimum(cm, jnp.where(avs[k] == mgv, ais[k], _N))
        nxtv = jnp.full((_L,), jnp.min(cm))
        return mgv, nxtv

    def init_dist(n_local):
        def init(j, carry):
            dist[pl.ds(j * _L, _L)] = big
            return carry
        lax.fori_loop(0, n_local // _L, init, 0, unroll=4)

    init_dist(_H)
    pt_base = half * _H
    myslot = s * 2 * _ROW
    pslot = (s ^ 1) * 2 * _ROW

    def stage1_iter(i, farv):
        cx = plsc.load_gather(xv, [farv])
        cy = plsc.load_gather(yv, [farv])
        cz = plsc.load_gather(zv, [farv])

        @pl.when(half == 0)
        def _():
            ivs = jnp.full((_L,), i, jnp.int32)
            plsc.store_scatter(outv, [ivs], farv, mask=lane0)
            plsc.store_scatter(s2x, [ivs], cx, mask=lane0)
            plsc.store_scatter(s2y, [ivs], cy, mask=lane0)
            plsc.store_scatter(s2z, [ivs], cz, mask=lane0)

        mgv, nxtv = scan_half(xv, yv, zv, pt_base, _H, cx, cy, cz)

        exw[pl.ds(0, _L)] = plsc.bitcast(mgv, jnp.int32)
        exw[pl.ds(_L, _L)] = nxtv
        parity = (i % 2) * _ROW
        pltpu.sync_copy(exw, shared.at[pl.ds(myslot + parity, _ROW)])
        plsc.subcore_barrier()
        pltpu.sync_copy(shared.at[pl.ds(pslot + parity, _ROW)], exr)
        pmv = plsc.bitcast(exr[pl.ds(0, _L)], jnp.float32)
        piv = exr[pl.ds(_L, _L)]
        take = (pmv > mgv) | ((pmv == mgv) & (piv < nxtv))
        return jnp.where(take, piv, nxtv)

    lax.fori_loop(0, 1024, stage1_iter, jnp.zeros((_L,), jnp.int32))

    @pl.when(half == 0)
    def _():
        def run_stage(px, py, pz, n, npoint, off, prev_off, sel):
            init_dist(n)

            def one_iter(i, farv):
                cx = plsc.load_gather(px, [farv])
                cy = plsc.load_gather(py, [farv])
                cz = plsc.load_gather(pz, [farv])
                iv = jnp.full((_L,), off + i, jnp.int32)
                val = plsc.load_gather(outv, [prev_off + farv])
                plsc.store_scatter(outv, [iv], val, mask=lane0)
                if sel is not None:
                    sx, sy, sz = sel
                    ivs = jnp.full((_L,), i, jnp.int32)
                    plsc.store_scatter(sx, [ivs], cx, mask=lane0)
                    plsc.store_scatter(sy, [ivs], cy, mask=lane0)
                    plsc.store_scatter(sz, [ivs], cz, mask=lane0)
                _, nxtv = scan_half(px, py, pz, 0, n, cx, cy, cz)
                return nxtv

            lax.fori_loop(0, npoint, one_iter, jnp.zeros((_L,), jnp.int32))

        run_stage(s2x, s2y, s2z, 1024, 256, 1024, 0, (s3x, s3y, s3z))
        run_stage(s3x, s3y, s3z, 256, 64, 1280, 1024, (s4x, s4y, s4z))
        run_stage(s4x, s4y, s4z, 64, 16, 1344, 1280, None)

        pltpu.sync_copy(outv, out.at[cloud])


@jax.jit
def kernel(x):
    xyz = x[..., :3]
    xs = xyz[..., 0]
    ys = xyz[..., 1]
    zs = xyz[..., 2]
    f = pl.kernel(
        _fps_body,
        out_type=jax.ShapeDtypeStruct((_B, _NOUT), jnp.int32),
        mesh=plsc.VectorSubcoreMesh(core_axis_name="c", subcore_axis_name="s"),
        compiler_params=pltpu.CompilerParams(needs_layout_passes=False),
        scratch_types=[
            pltpu.VMEM_SHARED((16 * 2 * _ROW,), jnp.int32),
            pltpu.VMEM((_N,), jnp.float32),
            pltpu.VMEM((_N,), jnp.float32),
            pltpu.VMEM((_N,), jnp.float32),
            pltpu.VMEM((_H,), jnp.float32),
            pltpu.VMEM((_ROW,), jnp.int32),
            pltpu.VMEM((_ROW,), jnp.int32),
            pltpu.VMEM((1024,), jnp.float32),
            pltpu.VMEM((1024,), jnp.float32),
            pltpu.VMEM((1024,), jnp.float32),
            pltpu.VMEM((256,), jnp.float32),
            pltpu.VMEM((256,), jnp.float32),
            pltpu.VMEM((256,), jnp.float32),
            pltpu.VMEM((64,), jnp.float32),
            pltpu.VMEM((64,), jnp.float32),
            pltpu.VMEM((64,), jnp.float32),
            pltpu.VMEM((_NOUT,), jnp.int32),
        ],
    )
    return f(xs, ys, zs)
